# Initial kernel scaffold; baseline (speedup 1.0000x reference)
import jax
import jax.numpy as jnp
from jax.experimental import pallas as pl


def kernel(query_points, key_features, W1, b1, W2, b2, W3, b3):
    raise NotImplementedError("write your pallas kernel here")


# speedup vs baseline: 4.2472x; 4.2472x over previous
import functools

import numpy as np
import jax
import jax.numpy as jnp
from jax import lax
from jax.experimental import pallas as pl
from jax.experimental.pallas import tpu as pltpu
from jax.experimental.pallas import tpu_sc as plsc

_RADIUS2 = np.float32(0.4 * 0.4)
_K = 64
_L = 16
_NC = 2
_NS = 16
_BIG = np.int32(2**30)


def _make_sc_ball_gather(B, N, C, NPAD, G):
    NW = _NC * _NS
    QW = N // NW

    mesh = plsc.VectorSubcoreMesh(core_axis_name="c", subcore_axis_name="s",
                                  num_cores=_NC, num_subcores=_NS)

    @functools.partial(
        pl.kernel,
        out_type=jax.ShapeDtypeStruct((B * N, _K, C), jnp.float32),
        mesh=mesh,
        compiler_params=pltpu.CompilerParams(needs_layout_passes=False,
                                             use_tc_tiling_on_sc=False),
        scratch_types=[
            pltpu.VMEM((N,), jnp.float32),
            pltpu.VMEM((N,), jnp.float32),
            pltpu.VMEM((N,), jnp.float32),
            pltpu.VMEM((N + _L,), jnp.float32),
            pltpu.VMEM((N + _L,), jnp.int32),
            pltpu.VMEM((G * _K,), jnp.int32),
            pltpu.VMEM((G, _K, C), jnp.float32),
            pltpu.SemaphoreType.DMA,
        ],
    )
    def sc_kernel(qp_hbm, feats_hbm, out_hbm, kx, ky, kz, cd2, cidx, gidx,
                  rows, sem):
        cid = lax.axis_index("c")
        sid = lax.axis_index("s")
        wid = sid * _NC + cid
        iota = lax.iota(jnp.int32, _L)
        lane0 = iota == 0
        infv = jnp.full((_L,), jnp.inf, jnp.float32)

        for b in range(B):
            pltpu.sync_copy(qp_hbm.at[b * 3 + 0], kx)
            pltpu.sync_copy(qp_hbm.at[b * 3 + 1], ky)
            pltpu.sync_copy(qp_hbm.at[b * 3 + 2], kz)
            base_row = b * NPAD
            pad_row = base_row + N
            qbase = wid * QW

            def group_body(g, _, b=b, base_row=base_row, pad_row=pad_row,
                           qbase=qbase):
                def query_body(qq, _, g=g):
                    n = qbase + g * G + qq
                    nv = jnp.full((_L,), n, jnp.int32)
                    qx = plsc.load_gather(kx, [nv])
                    qy = plsc.load_gather(ky, [nv])
                    qz = plsc.load_gather(kz, [nv])

                    def scan_chunk(j, cnt):
                        off = j * _L
                        dx = qx - kx[pl.ds(off, _L)]
                        dy = qy - ky[pl.ds(off, _L)]
                        dz = qz - kz[pl.ds(off, _L)]
                        d2 = dx * dx + dy * dy + dz * dz
                        m = d2 <= _RADIUS2
                        plsc.store_compressed(cd2.at[pl.ds(cnt, _L)], d2,
                                              mask=m)
                        plsc.store_compressed(
                            cidx.at[pl.ds(cnt, _L)],
                            iota + (off + base_row), mask=m)
                        return cnt + jnp.max(
                            plsc.all_reduce_population_count(m))

                    cnt = lax.fori_loop(0, N // _L, scan_chunk, jnp.int32(0))
                    cd2[pl.ds(cnt, _L)] = infv

                    padv = jnp.full((_L,), pad_row, jnp.int32)
                    for kk in range(_K // _L):
                        gidx[pl.ds(qq * _K + kk * _L, _L)] = padv

                    nsel = jnp.minimum(cnt, _K)
                    nchunks = (cnt + (_L - 1)) // _L

                    def extract(k_slot, _):
                        def minpass(c, mv):
                            v = cd2[pl.ds(c * _L, _L)]
                            return jnp.minimum(mv, jnp.min(v))

                        mval = lax.fori_loop(0, nchunks, minpass,
                                             jnp.float32(jnp.inf))
                        mvalv = jnp.full((_L,), mval, jnp.float32)

                        def pospass(c, pv):
                            v = cd2[pl.ds(c * _L, _L)]
                            pos = jnp.where(v == mvalv, c * _L + iota,
                                            jnp.full((_L,), _BIG, jnp.int32))
                            return jnp.minimum(pv, jnp.min(pos))

                        mpos = lax.fori_loop(0, nchunks, pospass, _BIG)
                        posv = jnp.full((_L,), mpos, jnp.int32)
                        chosen = plsc.load_gather(cidx, [posv])
                        plsc.store_scatter(
                            gidx,
                            [jnp.full((_L,), qq * _K + k_slot, jnp.int32)],
                            chosen, mask=lane0)
                        plsc.store_scatter(cd2, [posv], infv, mask=lane0)
                        return 0

                    lax.fori_loop(0, nsel, extract, 0)
                    return 0

                lax.fori_loop(0, G, query_body, 0)
                descs = [
                    pltpu.async_copy(
                        feats_hbm.at[gidx.at[pl.ds(qq * _K, _K)]],
                        rows.at[qq], sem)
                    for qq in range(G)
                ]
                for d in descs:
                    d.wait()
                out_base = b * N + qbase + g * G
                pltpu.sync_copy(rows, out_hbm.at[pl.ds(out_base, G)])
                return 0

            lax.fori_loop(0, QW // G, group_body, 0)

    return sc_kernel


def _gelu_exact(x):
    return x * 0.5 * (1.0 + lax.erf(x * np.float32(1.0 / np.sqrt(2.0))))


def _mlp_tc(flat, W1, b1, W2, b2, W3, b3, block_rows=512):
    R, F = flat.shape
    H = W1.shape[1]

    def body(x_ref, w1_ref, b1_ref, w2_ref, b2_ref, w3_ref, b3_ref, o_ref):
        h = jnp.dot(x_ref[...], w1_ref[...],
                    preferred_element_type=jnp.float32) + b1_ref[...]
        h = _gelu_exact(h)
        h = jnp.dot(h, w2_ref[...],
                    preferred_element_type=jnp.float32) + b2_ref[...]
        h = _gelu_exact(h)
        h = jnp.dot(h, w3_ref[...],
                    preferred_element_type=jnp.float32) + b3_ref[...]
        o_ref[...] = jnp.tanh(h)

    return pl.pallas_call(
        body,
        grid=(R // block_rows,),
        in_specs=[
            pl.BlockSpec((block_rows, F), lambda i: (i, 0)),
            pl.BlockSpec(W1.shape, lambda i: (0, 0)),
            pl.BlockSpec((1, W1.shape[1]), lambda i: (0, 0)),
            pl.BlockSpec(W2.shape, lambda i: (0, 0)),
            pl.BlockSpec((1, W2.shape[1]), lambda i: (0, 0)),
            pl.BlockSpec(W3.shape, lambda i: (0, 0)),
            pl.BlockSpec((1, W3.shape[1]), lambda i: (0, 0)),
        ],
        out_specs=pl.BlockSpec((block_rows, H), lambda i: (i, 0)),
        out_shape=jax.ShapeDtypeStruct((R, H), jnp.float32),
    )(flat, W1, b1.reshape(1, -1), W2, b2.reshape(1, -1), W3,
      b3.reshape(1, -1))


def kernel(query_points, key_features, W1, b1, W2, b2, W3, b3):
    B, N, C = key_features.shape
    NPAD = N + 8
    qp_t = jnp.transpose(query_points, (0, 2, 1)).reshape(B * 3, N)
    feats_flat = jnp.pad(key_features,
                         ((0, 0), (0, NPAD - N), (0, 0))).reshape(B * NPAD, C)
    sc = _make_sc_ball_gather(B, N, C, NPAD, G=16)
    gathered = sc(qp_t, feats_flat)
    flat = gathered.reshape(B * N, _K * C)
    out = _mlp_tc(flat, W1, b1, W2, b2, W3, b3)
    return out.reshape(B, N, W1.shape[1])
